# baseline (speedup 1.0000x reference)
import functools

import jax
import jax.numpy as jnp
import numpy as np
from jax import lax
from jax.experimental import pallas as pl
from jax.experimental.pallas import tpu as pltpu
from jax.experimental.pallas import tpu_sc as plsc

VOCAB = 1000000
EMBED_DIM = 32
HIDDEN_DIM = 64
OUTPUT_SIZE = 32
BATCH = 16384
HIST = 50

NUM_CORES = 2
NUM_SUBCORES = 16
NUM_WORKERS = NUM_CORES * NUM_SUBCORES

ROWS_W = BATCH // NUM_WORKERS
IDX_W = ROWS_W * HIST
GROUPS_PER_CHUNK = 8
CHUNK = GROUPS_PER_CHUNK * HIST
NCHUNK = IDX_W // CHUNK
SUB = 80
NSUB = CHUNK // SUB
LANES = 16

_DEINT_PERM = np.concatenate([np.arange(0, EMBED_DIM, 2),
                              np.arange(1, EMBED_DIM, 2)])


def _sc_gather_pool(x_flat, table_bf16):
    mesh = plsc.VectorSubcoreMesh(core_axis_name="c", subcore_axis_name="s")

    @functools.partial(
        pl.kernel,
        out_type=jax.ShapeDtypeStruct((BATCH, EMBED_DIM), jnp.float32),
        mesh=mesh,
        compiler_params=pltpu.CompilerParams(
            use_tc_tiling_on_sc=False, needs_layout_passes=False),
        scratch_types=[
            pltpu.VMEM((IDX_W,), jnp.int32),
            pltpu.VMEM((CHUNK, EMBED_DIM // 2), jnp.float32),
            pltpu.VMEM((ROWS_W, EMBED_DIM), jnp.float32),
        ],
    )
    def sc_kernel(x_hbm, table_hbm, out_hbm, idx_v, rows_v, pooled_v):
        wid = lax.axis_index("s") * NUM_CORES + lax.axis_index("c")
        pltpu.sync_copy(x_hbm.at[pl.ds(wid * IDX_W, IDX_W)], idx_v)

        @pl.loop(0, NCHUNK)
        def _(c):
            ibase = c * CHUNK
            for k in range(NSUB):
                pltpu.sync_copy(
                    table_hbm.at[idx_v.at[pl.ds(ibase + k * SUB, SUB)]],
                    rows_v.at[pl.ds(k * SUB, SUB)],
                )

            @pl.loop(0, GROUPS_PER_CHUNK)
            def _(g):
                rbase = g * HIST
                acc_a, acc_b = plsc.unpack(
                    plsc.bitcast(rows_v[rbase, :], jnp.bfloat16),
                    format=plsc.PackFormat.INTERLEAVED)
                for j in range(1, HIST):
                    a, b = plsc.unpack(
                        plsc.bitcast(rows_v[rbase + j, :], jnp.bfloat16),
                        format=plsc.PackFormat.INTERLEAVED)
                    acc_a = acc_a + a
                    acc_b = acc_b + b
                row = c * GROUPS_PER_CHUNK + g
                pooled_v[row, pl.ds(0, LANES)] = acc_a
                pooled_v[row, pl.ds(LANES, LANES)] = acc_b

        pltpu.sync_copy(pooled_v, out_hbm.at[pl.ds(wid * ROWS_W, ROWS_W)])

    return sc_kernel(x_flat, table_bf16)


BLKV = 2048
EIGHTH = BLKV // 8
NBLK = (VOCAB + BLKV - 1) // BLKV
VOCAB_PAD = NBLK * BLKV
WORDS = EMBED_DIM // 2
OUT_ROWS = VOCAB_PAD * WORDS // 128


def _prep_body(tt_ref, o_ref):
    t = tt_ref[...]
    u = jax.lax.bitcast_convert_type(t, jnp.uint32)
    r = ((u + jnp.uint32(0x7FFF) + ((u >> 16) & jnp.uint32(1)))
         & jnp.uint32(0xFFFF0000))
    word = (r[:WORDS, :] >> 16) | (r[WORDS:, :] & jnp.uint32(0xFFFF0000))
    w = jax.lax.bitcast_convert_type(word, jnp.float32)
    for k in range(8):
        o_ref[:, k * WORDS:(k + 1) * WORDS] = (
            w[:, k * EIGHTH:(k + 1) * EIGHTH].T)


def _tc_prep(table):
    return pl.pallas_call(
        _prep_body,
        grid=(NBLK,),
        in_specs=[pl.BlockSpec((EMBED_DIM, BLKV), lambda i: (0, i))],
        out_specs=pl.BlockSpec((EIGHTH, 128), lambda i: (i, 0)),
        out_shape=jax.ShapeDtypeStruct((OUT_ROWS, 128), jnp.float32),
        compiler_params=pltpu.CompilerParams(
            dimension_semantics=("parallel",)),
    )(table.T)


def _permute_indices(x_flat):
    return ((x_flat & ~jnp.int32(2047))
            + ((x_flat & jnp.int32(255)) << 3)
            + ((x_flat >> 8) & jnp.int32(7)))


def _mlp_body(p_ref, w1t_ref, b1_ref, w2t_ref, b2_ref, o_ref):
    p = p_ref[...] * jnp.float32(1.0 / HIST)
    h = jnp.dot(p, w1t_ref[...], preferred_element_type=jnp.float32)
    h = jnp.maximum(h + b1_ref[...], 0.0)
    o = jnp.dot(h, w2t_ref[...], preferred_element_type=jnp.float32)
    o_ref[...] = o + b2_ref[...]


def _tc_mlp(pooled_sums, W1, b1, W2, b2):
    blk = 2048
    grid = (BATCH // blk,)
    return pl.pallas_call(
        _mlp_body,
        grid=grid,
        in_specs=[
            pl.BlockSpec((blk, EMBED_DIM), lambda i: (i, 0)),
            pl.BlockSpec((EMBED_DIM, HIDDEN_DIM), lambda i: (0, 0)),
            pl.BlockSpec((1, HIDDEN_DIM), lambda i: (0, 0)),
            pl.BlockSpec((HIDDEN_DIM, OUTPUT_SIZE), lambda i: (0, 0)),
            pl.BlockSpec((1, OUTPUT_SIZE), lambda i: (0, 0)),
        ],
        out_specs=pl.BlockSpec((blk, OUTPUT_SIZE), lambda i: (i, 0)),
        out_shape=jax.ShapeDtypeStruct((BATCH, OUTPUT_SIZE), jnp.float32),
    )(
        pooled_sums,
        W1.T,
        b1.reshape(1, HIDDEN_DIM),
        W2.T,
        b2.reshape(1, OUTPUT_SIZE),
    )


def kernel(x, table, W1, b1, W2, b2):
    sums = _sc_gather_pool(
        _permute_indices(x.reshape(-1)),
        _tc_prep(table).reshape(VOCAB_PAD, EMBED_DIM // 2))
    return _tc_mlp(sums, W1, b1, W2, b2)


# speedup vs baseline: 1.2461x; 1.2461x over previous
import functools

import jax
import jax.numpy as jnp
import numpy as np
from jax import lax
from jax.experimental import pallas as pl
from jax.experimental.pallas import tpu as pltpu
from jax.experimental.pallas import tpu_sc as plsc

VOCAB = 1000000
EMBED_DIM = 32
HIDDEN_DIM = 64
OUTPUT_SIZE = 32
BATCH = 16384
HIST = 50

NUM_CORES = 2
NUM_SUBCORES = 16
NUM_WORKERS = NUM_CORES * NUM_SUBCORES

ROWS_W = BATCH // NUM_WORKERS
IDX_W = ROWS_W * HIST
GROUPS_PER_CHUNK = 8
CHUNK = GROUPS_PER_CHUNK * HIST
NCHUNK = IDX_W // CHUNK
SUB = 80
NSUB = CHUNK // SUB
LANES = 16

_DEINT_PERM = np.concatenate([np.arange(0, EMBED_DIM, 2),
                              np.arange(1, EMBED_DIM, 2)])


def _sc_gather_pool(x_flat, table_bf16):
    mesh = plsc.VectorSubcoreMesh(core_axis_name="c", subcore_axis_name="s")

    @functools.partial(
        pl.kernel,
        out_type=jax.ShapeDtypeStruct((BATCH, EMBED_DIM), jnp.float32),
        mesh=mesh,
        compiler_params=pltpu.CompilerParams(
            use_tc_tiling_on_sc=False, needs_layout_passes=False),
        scratch_types=[
            pltpu.VMEM((IDX_W,), jnp.int32),
            pltpu.VMEM((CHUNK, EMBED_DIM // 2), jnp.float32),
            pltpu.VMEM((ROWS_W, EMBED_DIM), jnp.float32),
        ],
    )
    def sc_kernel(x_hbm, table_hbm, out_hbm, idx_v, rows_v, pooled_v):
        wid = lax.axis_index("s") * NUM_CORES + lax.axis_index("c")
        pltpu.sync_copy(x_hbm.at[pl.ds(wid * IDX_W, IDX_W)], idx_v)

        @pl.loop(0, NCHUNK)
        def _(c):
            ibase = c * CHUNK
            for k in range(NSUB):
                pltpu.sync_copy(
                    table_hbm.at[idx_v.at[pl.ds(ibase + k * SUB, SUB)]],
                    rows_v.at[pl.ds(k * SUB, SUB)],
                )

            @pl.loop(0, GROUPS_PER_CHUNK)
            def _(g):
                rbase = g * HIST
                acc_a, acc_b = plsc.unpack(
                    plsc.bitcast(rows_v[rbase, :], jnp.bfloat16),
                    format=plsc.PackFormat.INTERLEAVED)
                for j in range(1, HIST):
                    a, b = plsc.unpack(
                        plsc.bitcast(rows_v[rbase + j, :], jnp.bfloat16),
                        format=plsc.PackFormat.INTERLEAVED)
                    acc_a = acc_a + a
                    acc_b = acc_b + b
                row = c * GROUPS_PER_CHUNK + g
                pooled_v[row, pl.ds(0, LANES)] = acc_a
                pooled_v[row, pl.ds(LANES, LANES)] = acc_b

        pltpu.sync_copy(pooled_v, out_hbm.at[pl.ds(wid * ROWS_W, ROWS_W)])

    return sc_kernel(x_flat, table_bf16)


BLKV = 8192
EIGHTH = BLKV // 8
NBLK = (VOCAB + BLKV - 1) // BLKV
VOCAB_PAD = NBLK * BLKV
WORDS = EMBED_DIM // 2
OUT_ROWS = VOCAB_PAD * WORDS // 128


def _prep_body(tt_ref, o_ref):
    t = tt_ref[...]
    u = jax.lax.bitcast_convert_type(t, jnp.uint32)
    r = ((u + jnp.uint32(0x7FFF) + ((u >> 16) & jnp.uint32(1)))
         & jnp.uint32(0xFFFF0000))
    word = (r[:WORDS, :] >> 16) | (r[WORDS:, :] & jnp.uint32(0xFFFF0000))
    w = jax.lax.bitcast_convert_type(word, jnp.float32)
    for k in range(8):
        o_ref[:, k * WORDS:(k + 1) * WORDS] = (
            w[:, k * EIGHTH:(k + 1) * EIGHTH].T)


def _tc_prep(table):
    return pl.pallas_call(
        _prep_body,
        grid=(NBLK,),
        in_specs=[pl.BlockSpec((EMBED_DIM, BLKV), lambda i: (0, i))],
        out_specs=pl.BlockSpec((EIGHTH, 128), lambda i: (i, 0)),
        out_shape=jax.ShapeDtypeStruct((OUT_ROWS, 128), jnp.float32),
        compiler_params=pltpu.CompilerParams(
            dimension_semantics=("parallel",)),
    )(table.T)


_EIGHTH_BITS = EIGHTH.bit_length() - 1


def _permute_indices(x_flat):
    return ((x_flat & ~jnp.int32(BLKV - 1))
            + ((x_flat & jnp.int32(EIGHTH - 1)) << 3)
            + ((x_flat >> _EIGHTH_BITS) & jnp.int32(7)))


def _mlp_body(p_ref, w1t_ref, b1_ref, w2t_ref, b2_ref, o_ref):
    p = p_ref[...] * jnp.float32(1.0 / HIST)
    h = jnp.dot(p, w1t_ref[...], preferred_element_type=jnp.float32)
    h = jnp.maximum(h + b1_ref[...], 0.0)
    o = jnp.dot(h, w2t_ref[...], preferred_element_type=jnp.float32)
    o_ref[...] = o + b2_ref[...]


def _tc_mlp(pooled_sums, W1, b1, W2, b2):
    blk = 2048
    grid = (BATCH // blk,)
    return pl.pallas_call(
        _mlp_body,
        grid=grid,
        in_specs=[
            pl.BlockSpec((blk, EMBED_DIM), lambda i: (i, 0)),
            pl.BlockSpec((EMBED_DIM, HIDDEN_DIM), lambda i: (0, 0)),
            pl.BlockSpec((1, HIDDEN_DIM), lambda i: (0, 0)),
            pl.BlockSpec((HIDDEN_DIM, OUTPUT_SIZE), lambda i: (0, 0)),
            pl.BlockSpec((1, OUTPUT_SIZE), lambda i: (0, 0)),
        ],
        out_specs=pl.BlockSpec((blk, OUTPUT_SIZE), lambda i: (i, 0)),
        out_shape=jax.ShapeDtypeStruct((BATCH, OUTPUT_SIZE), jnp.float32),
    )(
        pooled_sums,
        W1.T,
        b1.reshape(1, HIDDEN_DIM),
        W2.T,
        b2.reshape(1, OUTPUT_SIZE),
    )


def kernel(x, table, W1, b1, W2, b2):
    sums = _sc_gather_pool(
        _permute_indices(x.reshape(-1)),
        _tc_prep(table).reshape(VOCAB_PAD, EMBED_DIM // 2))
    return _tc_mlp(sums, W1, b1, W2, b2)


# speedup vs baseline: 1.2611x; 1.0120x over previous
import functools

import jax
import jax.numpy as jnp
import numpy as np
from jax import lax
from jax.experimental import pallas as pl
from jax.experimental.pallas import tpu as pltpu
from jax.experimental.pallas import tpu_sc as plsc

VOCAB = 1000000
EMBED_DIM = 32
HIDDEN_DIM = 64
OUTPUT_SIZE = 32
BATCH = 16384
HIST = 50

NUM_CORES = 2
NUM_SUBCORES = 16
NUM_WORKERS = NUM_CORES * NUM_SUBCORES

ROWS_W = BATCH // NUM_WORKERS
IDX_W = ROWS_W * HIST
GROUPS_PER_CHUNK = 8
CHUNK = GROUPS_PER_CHUNK * HIST
NCHUNK = IDX_W // CHUNK
SUB = 80
NSUB = CHUNK // SUB
LANES = 16

_DEINT_PERM = np.concatenate([np.arange(0, EMBED_DIM, 2),
                              np.arange(1, EMBED_DIM, 2)])


def _sc_gather_pool(x_flat, table_bf16):
    mesh = plsc.VectorSubcoreMesh(core_axis_name="c", subcore_axis_name="s")

    @functools.partial(
        pl.kernel,
        out_type=jax.ShapeDtypeStruct((BATCH, EMBED_DIM), jnp.float32),
        mesh=mesh,
        compiler_params=pltpu.CompilerParams(
            use_tc_tiling_on_sc=False, needs_layout_passes=False),
        scratch_types=[
            pltpu.VMEM((IDX_W,), jnp.int32),
            pltpu.VMEM((CHUNK, EMBED_DIM // 2), jnp.float32),
            pltpu.VMEM((ROWS_W, EMBED_DIM), jnp.float32),
        ],
    )
    def sc_kernel(x_hbm, table_hbm, out_hbm, idx_v, rows_v, pooled_v):
        wid = lax.axis_index("s") * NUM_CORES + lax.axis_index("c")
        pltpu.sync_copy(x_hbm.at[pl.ds(wid * IDX_W, IDX_W)], idx_v)

        @pl.loop(0, NCHUNK)
        def _(c):
            ibase = c * CHUNK
            for k in range(NSUB):
                pltpu.sync_copy(
                    table_hbm.at[idx_v.at[pl.ds(ibase + k * SUB, SUB)]],
                    rows_v.at[pl.ds(k * SUB, SUB)],
                )

            @pl.loop(0, GROUPS_PER_CHUNK)
            def _(g):
                rbase = g * HIST
                acc_a, acc_b = plsc.unpack(
                    plsc.bitcast(rows_v[rbase, :], jnp.bfloat16),
                    format=plsc.PackFormat.INTERLEAVED)
                for j in range(1, HIST):
                    a, b = plsc.unpack(
                        plsc.bitcast(rows_v[rbase + j, :], jnp.bfloat16),
                        format=plsc.PackFormat.INTERLEAVED)
                    acc_a = acc_a + a
                    acc_b = acc_b + b
                row = c * GROUPS_PER_CHUNK + g
                pooled_v[row, pl.ds(0, LANES)] = acc_a
                pooled_v[row, pl.ds(LANES, LANES)] = acc_b

        pltpu.sync_copy(pooled_v, out_hbm.at[pl.ds(wid * ROWS_W, ROWS_W)])

    return sc_kernel(x_flat, table_bf16)


BLKV = 32768
EIGHTH = BLKV // 8
NBLK = (VOCAB + BLKV - 1) // BLKV
VOCAB_PAD = NBLK * BLKV
WORDS = EMBED_DIM // 2
OUT_ROWS = VOCAB_PAD * WORDS // 128


def _prep_body(tt_ref, o_ref):
    t = tt_ref[...]
    u = jax.lax.bitcast_convert_type(t, jnp.uint32)
    r = ((u + jnp.uint32(0x7FFF) + ((u >> 16) & jnp.uint32(1)))
         & jnp.uint32(0xFFFF0000))
    word = (r[:WORDS, :] >> 16) | (r[WORDS:, :] & jnp.uint32(0xFFFF0000))
    w = jax.lax.bitcast_convert_type(word, jnp.float32)
    for k in range(8):
        o_ref[:, k * WORDS:(k + 1) * WORDS] = (
            w[:, k * EIGHTH:(k + 1) * EIGHTH].T)


def _tc_prep(table):
    return pl.pallas_call(
        _prep_body,
        grid=(NBLK,),
        in_specs=[pl.BlockSpec((EMBED_DIM, BLKV), lambda i: (0, i))],
        out_specs=pl.BlockSpec((EIGHTH, 128), lambda i: (i, 0)),
        out_shape=jax.ShapeDtypeStruct((OUT_ROWS, 128), jnp.float32),
        compiler_params=pltpu.CompilerParams(
            dimension_semantics=("parallel",)),
    )(table.T)


_EIGHTH_BITS = EIGHTH.bit_length() - 1


def _permute_indices(x_flat):
    return ((x_flat & ~jnp.int32(BLKV - 1))
            + ((x_flat & jnp.int32(EIGHTH - 1)) << 3)
            + ((x_flat >> _EIGHTH_BITS) & jnp.int32(7)))


def _mlp_body(p_ref, w1t_ref, b1_ref, w2t_ref, b2_ref, o_ref):
    p = p_ref[...] * jnp.float32(1.0 / HIST)
    h = jnp.dot(p, w1t_ref[...], preferred_element_type=jnp.float32)
    h = jnp.maximum(h + b1_ref[...], 0.0)
    o = jnp.dot(h, w2t_ref[...], preferred_element_type=jnp.float32)
    o_ref[...] = o + b2_ref[...]


def _tc_mlp(pooled_sums, W1, b1, W2, b2):
    blk = 2048
    grid = (BATCH // blk,)
    return pl.pallas_call(
        _mlp_body,
        grid=grid,
        in_specs=[
            pl.BlockSpec((blk, EMBED_DIM), lambda i: (i, 0)),
            pl.BlockSpec((EMBED_DIM, HIDDEN_DIM), lambda i: (0, 0)),
            pl.BlockSpec((1, HIDDEN_DIM), lambda i: (0, 0)),
            pl.BlockSpec((HIDDEN_DIM, OUTPUT_SIZE), lambda i: (0, 0)),
            pl.BlockSpec((1, OUTPUT_SIZE), lambda i: (0, 0)),
        ],
        out_specs=pl.BlockSpec((blk, OUTPUT_SIZE), lambda i: (i, 0)),
        out_shape=jax.ShapeDtypeStruct((BATCH, OUTPUT_SIZE), jnp.float32),
    )(
        pooled_sums,
        W1.T,
        b1.reshape(1, HIDDEN_DIM),
        W2.T,
        b2.reshape(1, OUTPUT_SIZE),
    )


def kernel(x, table, W1, b1, W2, b2):
    sums = _sc_gather_pool(
        _permute_indices(x.reshape(-1)),
        _tc_prep(table).reshape(VOCAB_PAD, EMBED_DIM // 2))
    return _tc_mlp(sums, W1, b1, W2, b2)


# speedup vs baseline: 1.8308x; 1.4517x over previous
import functools

import jax
import jax.numpy as jnp
import numpy as np
from jax import lax
from jax.experimental import pallas as pl
from jax.experimental.pallas import tpu as pltpu
from jax.experimental.pallas import tpu_sc as plsc

VOCAB = 1000000
EMBED_DIM = 32
HIDDEN_DIM = 64
OUTPUT_SIZE = 32
BATCH = 16384
HIST = 50

NUM_CORES = 2
NUM_SUBCORES = 16
NUM_WORKERS = NUM_CORES * NUM_SUBCORES

ROWS_W = BATCH // NUM_WORKERS
IDX_W = ROWS_W * HIST
GROUPS_PER_CHUNK = 8
CHUNK = GROUPS_PER_CHUNK * HIST
NCHUNK = IDX_W // CHUNK
SUB = 80
NSUB = CHUNK // SUB
LANES = 16

_DEINT_PERM = np.concatenate([np.arange(0, EMBED_DIM, 2),
                              np.arange(1, EMBED_DIM, 2)])


def _sc_gather_pool(x_flat, table_bf16):
    mesh = plsc.VectorSubcoreMesh(core_axis_name="c", subcore_axis_name="s")

    @functools.partial(
        pl.kernel,
        out_type=jax.ShapeDtypeStruct((BATCH, EMBED_DIM), jnp.float32),
        mesh=mesh,
        compiler_params=pltpu.CompilerParams(
            use_tc_tiling_on_sc=False, needs_layout_passes=False),
        scratch_types=[
            pltpu.VMEM((IDX_W,), jnp.int32),
            pltpu.VMEM((CHUNK, EMBED_DIM // 2), jnp.float32),
            pltpu.VMEM((CHUNK, EMBED_DIM // 2), jnp.float32),
            pltpu.VMEM((ROWS_W, EMBED_DIM), jnp.float32),
            pltpu.SemaphoreType.DMA,
            pltpu.SemaphoreType.DMA,
        ],
    )
    def sc_kernel(x_hbm, table_hbm, out_hbm, idx_v, rows0_v, rows1_v,
                  pooled_v, sem0, sem1):
        wid = lax.axis_index("s") * NUM_CORES + lax.axis_index("c")
        pltpu.sync_copy(x_hbm.at[pl.ds(wid * IDX_W, IDX_W)], idx_v)

        def gather_descs(c, rows_b, sem):
            ibase = c * CHUNK
            return [
                pltpu.make_async_copy(
                    table_hbm.at[idx_v.at[pl.ds(ibase + k * SUB, SUB)]],
                    rows_b.at[pl.ds(k * SUB, SUB)], sem)
                for k in range(NSUB)
            ]

        def fire(c, rows_b, sem):
            for d in gather_descs(c, rows_b, sem):
                d.start()

        def wait_chunk(c, rows_b, sem):
            for d in gather_descs(c, rows_b, sem):
                d.wait()

        def accum(c, rows_b):
            @pl.loop(0, GROUPS_PER_CHUNK)
            def _(g):
                rbase = g * HIST
                acc_a, acc_b = plsc.unpack(
                    plsc.bitcast(rows_b[rbase, :], jnp.bfloat16),
                    format=plsc.PackFormat.INTERLEAVED)
                for j in range(1, HIST):
                    a, b = plsc.unpack(
                        plsc.bitcast(rows_b[rbase + j, :], jnp.bfloat16),
                        format=plsc.PackFormat.INTERLEAVED)
                    acc_a = acc_a + a
                    acc_b = acc_b + b
                row = c * GROUPS_PER_CHUNK + g
                pooled_v[row, pl.ds(0, LANES)] = acc_a
                pooled_v[row, pl.ds(LANES, LANES)] = acc_b

        fire(0, rows0_v, sem0)

        @pl.loop(0, NCHUNK // 2)
        def _(h):
            c0 = h * 2
            wait_chunk(c0, rows0_v, sem0)
            fire(c0 + 1, rows1_v, sem1)
            accum(c0, rows0_v)
            wait_chunk(c0 + 1, rows1_v, sem1)
            fire(lax.rem(c0 + 2, NCHUNK), rows0_v, sem0)
            accum(c0 + 1, rows1_v)

        wait_chunk(0, rows0_v, sem0)
        pltpu.sync_copy(pooled_v, out_hbm.at[pl.ds(wid * ROWS_W, ROWS_W)])

    return sc_kernel(x_flat, table_bf16)


BLKV = 32768
EIGHTH = BLKV // 8
NBLK = (VOCAB + BLKV - 1) // BLKV
VOCAB_PAD = NBLK * BLKV
WORDS = EMBED_DIM // 2
OUT_ROWS = VOCAB_PAD * WORDS // 128


def _prep_body(tt_ref, o_ref):
    t = tt_ref[...]
    u = jax.lax.bitcast_convert_type(t, jnp.uint32)
    r = ((u + jnp.uint32(0x7FFF) + ((u >> 16) & jnp.uint32(1)))
         & jnp.uint32(0xFFFF0000))
    word = (r[:WORDS, :] >> 16) | (r[WORDS:, :] & jnp.uint32(0xFFFF0000))
    w = jax.lax.bitcast_convert_type(word, jnp.float32)
    for k in range(8):
        o_ref[:, k * WORDS:(k + 1) * WORDS] = (
            w[:, k * EIGHTH:(k + 1) * EIGHTH].T)


def _tc_prep(table):
    return pl.pallas_call(
        _prep_body,
        grid=(NBLK,),
        in_specs=[pl.BlockSpec((EMBED_DIM, BLKV), lambda i: (0, i))],
        out_specs=pl.BlockSpec((EIGHTH, 128), lambda i: (i, 0)),
        out_shape=jax.ShapeDtypeStruct((OUT_ROWS, 128), jnp.float32),
        compiler_params=pltpu.CompilerParams(
            dimension_semantics=("parallel",)),
    )(table.T)


_EIGHTH_BITS = EIGHTH.bit_length() - 1


def _permute_indices(x_flat):
    return ((x_flat & ~jnp.int32(BLKV - 1))
            + ((x_flat & jnp.int32(EIGHTH - 1)) << 3)
            + ((x_flat >> _EIGHTH_BITS) & jnp.int32(7)))


def _mlp_body(p_ref, w1t_ref, b1_ref, w2t_ref, b2_ref, o_ref):
    p = p_ref[...] * jnp.float32(1.0 / HIST)
    h = jnp.dot(p, w1t_ref[...], preferred_element_type=jnp.float32)
    h = jnp.maximum(h + b1_ref[...], 0.0)
    o = jnp.dot(h, w2t_ref[...], preferred_element_type=jnp.float32)
    o_ref[...] = o + b2_ref[...]


def _tc_mlp(pooled_sums, W1, b1, W2, b2):
    blk = 2048
    grid = (BATCH // blk,)
    return pl.pallas_call(
        _mlp_body,
        grid=grid,
        in_specs=[
            pl.BlockSpec((blk, EMBED_DIM), lambda i: (i, 0)),
            pl.BlockSpec((EMBED_DIM, HIDDEN_DIM), lambda i: (0, 0)),
            pl.BlockSpec((1, HIDDEN_DIM), lambda i: (0, 0)),
            pl.BlockSpec((HIDDEN_DIM, OUTPUT_SIZE), lambda i: (0, 0)),
            pl.BlockSpec((1, OUTPUT_SIZE), lambda i: (0, 0)),
        ],
        out_specs=pl.BlockSpec((blk, OUTPUT_SIZE), lambda i: (i, 0)),
        out_shape=jax.ShapeDtypeStruct((BATCH, OUTPUT_SIZE), jnp.float32),
    )(
        pooled_sums,
        W1.T,
        b1.reshape(1, HIDDEN_DIM),
        W2.T,
        b2.reshape(1, OUTPUT_SIZE),
    )


def kernel(x, table, W1, b1, W2, b2):
    sums = _sc_gather_pool(
        _permute_indices(x.reshape(-1)),
        _tc_prep(table).reshape(VOCAB_PAD, EMBED_DIM // 2))
    return _tc_mlp(sums, W1, b1, W2, b2)
